# baseline (speedup 1.0000x reference)
import functools

import jax
import jax.numpy as jnp
from jax import lax
from jax.experimental import pallas as pl
from jax.experimental.pallas import tpu as pltpu
from jax.experimental.pallas import tpu_sc as plsc

B = 128
S = 8192
H = S // 2
L = 16
NBINS = 256


def _body(tok_hbm, sc_hbm, mask_hbm, out_hbm,
          sc_h, mask_h, key_a, key_b, val_a, val_b,
          t0, t1, t2, t3, sums_s, g_s, num_workers):
    lane = lax.broadcasted_iota(jnp.int32, (L,), 0)
    wid = lax.axis_index("s") * 2 + lax.axis_index("c")
    rows_per_w = B // num_workers
    hists = (t0, t1, t2, t3)
    ctrs = (t3, t2, t1, t0)

    def zero_tables():
        @plsc.parallel_loop(0, NBINS, unroll=8)
        def _zt(d):
            z = jnp.zeros((L,), jnp.int32)
            for h in hists:
                h[pl.ds(d * L, L)] = z

    zero_tables()

    def do_row(r, carry_row):
        row = wid * rows_per_w + r
        pltpu.sync_copy(tok_hbm.at[row], key_b)

        def scan_half(carry, half_base):
            pltpu.sync_copy(sc_hbm.at[row, pl.ds(half_base, H)], sc_h)
            pltpu.sync_copy(mask_hbm.at[row, pl.ds(half_base, H)], mask_h)

            @plsc.parallel_loop(0, H // L, unroll=4, carry=carry)
            def counts(v, c):
                off_k, off_t = c
                lbase = H - L * (v + 1)
                scv = jnp.flip(sc_h[pl.ds(lbase, L)], axis=0)
                mkv = jnp.flip(mask_h[pl.ds(lbase, L)], axis=0)
                tkv = jnp.flip(key_b[pl.ds(half_base + lbase, L)], axis=0)
                keep = mkv != 0
                bits = lax.bitcast_convert_type(scv, jnp.int32)
                pos_key = jnp.bitwise_and(jnp.bitwise_not(bits),
                                          jnp.int32(0x7FFFFFFF))
                key = jnp.where(bits < 0, bits, pos_key)
                ki = plsc.cumsum(keep.astype(jnp.int32))
                pos_k = off_k + ki - 1
                plsc.store_scatter(key_a, [pos_k], key, mask=keep)
                plsc.store_scatter(val_a, [pos_k], tkv, mask=keep)
                nk = ki[L - 1]
                drop = jnp.logical_not(keep)
                pos_t = off_t + lane - ki
                plsc.store_scatter(val_b, [jnp.int32(S - 1) - pos_t], tkv,
                                   mask=drop)
                return (off_k + nk, off_t + (L - nk))

            return counts

        carry = scan_half((jnp.int32(0), jnp.int32(0)), H)
        m_cnt, d_cnt = scan_half(carry, 0)

        qc = (m_cnt + 63) // 64
        chunk = 4 * qc

        def do_pass(kb_s, vb_s, kb_d, vb_d, shift, write_keys=True):
            lane_c = lane * chunk
            qbase = [lane_c + q * qc for q in range(4)]

            @plsc.parallel_loop(0, qc, unroll=2)
            def hist_loop(t):
                ones = jnp.ones((L,), jnp.int32)
                for q in range(4):
                    idx = qbase[q] + t
                    valid = idx < m_cnt
                    k = plsc.load_gather(kb_s, [idx], mask=valid)
                    d16 = jnp.bitwise_and(lax.shift_right_logical(k, shift),
                                          jnp.int32(NBINS - 1)) * L + lane
                    plsc.addupdate_scatter(hists[q], [d16], ones,
                                           mask=valid)

            @plsc.parallel_loop(0, NBINS, unroll=4)
            def presum_loop(d):
                hv = [h[pl.ds(d * L, L)] for h in hists]
                tot = hv[0] + hv[1] + hv[2] + hv[3]
                cs = plsc.cumsum(tot)
                excl = cs - tot
                b1 = excl + hv[0]
                b2 = b1 + hv[1]
                b3 = b2 + hv[2]
                sums_s[d] = cs[L - 1]
                ctrs[0][pl.ds(d * L, L)] = excl
                ctrs[1][pl.ds(d * L, L)] = b1
                ctrs[2][pl.ds(d * L, L)] = b2
                ctrs[3][pl.ds(d * L, L)] = b3

            @plsc.parallel_loop(0, NBINS, unroll=8, carry=jnp.int32(0))
            def g_loop(d, g):
                g_s[d] = g
                return g + sums_s[d]

            @plsc.parallel_loop(0, NBINS, unroll=4)
            def addg_loop(d):
                gv = jnp.full((L,), g_s[d], jnp.int32)
                for c in ctrs:
                    c[pl.ds(d * L, L)] = c[pl.ds(d * L, L)] + gv

            def scat_body(t2x, c):
                for tt in range(2):
                    t = t2x * 2 + tt
                    for q in range(4):
                        idx = qbase[q] + t
                        valid = idx < m_cnt
                        k = plsc.load_gather(kb_s, [idx], mask=valid)
                        val = plsc.load_gather(vb_s, [idx], mask=valid)
                        d16 = jnp.bitwise_and(
                            lax.shift_right_logical(k, shift),
                            jnp.int32(NBINS - 1)) * L + lane
                        pos = plsc.load_gather(ctrs[q], [d16], mask=valid)
                        if write_keys:
                            plsc.store_scatter(kb_d, [pos], k, mask=valid)
                        plsc.store_scatter(vb_d, [pos], val, mask=valid)
                        plsc.store_scatter(ctrs[q], [d16], pos + 1,
                                           mask=valid)
                return c
            lax.fori_loop(0, (qc + 1) // 2, scat_body, 0)

            zero_tables()

        do_pass(key_a, val_a, key_b, val_b, 0)
        do_pass(key_b, val_b, key_a, val_a, 8)
        do_pass(key_a, val_a, key_b, val_b, 16)
        do_pass(key_b, val_b, key_a, val_a, 24, write_keys=False)

        n_tail = (d_cnt + L - 1) // L

        @plsc.parallel_loop(0, n_tail, unroll=4)
        def tail_loop(j):
            t = jnp.flip(val_b[pl.ds(S - L * (j + 1), L)], axis=0)
            dst = m_cnt + j * L + lane
            plsc.store_scatter(val_a, [dst], t, mask=dst < jnp.int32(S))

        pltpu.sync_copy(val_a, out_hbm.at[row])
        return carry_row

    lax.fori_loop(0, rows_per_w, do_row, 0)


@jax.jit
def kernel(query_tokens, rag_scores, attention_mask):
    info = plsc.get_sparse_core_info()
    num_workers = info.num_cores * info.num_subcores
    mesh = plsc.VectorSubcoreMesh(core_axis_name="c", subcore_axis_name="s")
    body = functools.partial(_body, num_workers=num_workers)
    fn = pl.kernel(
        body,
        out_type=jax.ShapeDtypeStruct((B, S), jnp.int32),
        mesh=mesh,
        compiler_params=pltpu.CompilerParams(needs_layout_passes=False),
        scratch_types=[
            pltpu.VMEM((H,), jnp.float32),
            pltpu.VMEM((H,), jnp.int32),
            pltpu.VMEM((S,), jnp.int32),
            pltpu.VMEM((S,), jnp.int32),
            pltpu.VMEM((S,), jnp.int32),
            pltpu.VMEM((S,), jnp.int32),
            pltpu.VMEM((NBINS * L,), jnp.int32),
            pltpu.VMEM((NBINS * L,), jnp.int32),
            pltpu.VMEM((NBINS * L,), jnp.int32),
            pltpu.VMEM((NBINS * L,), jnp.int32),
            pltpu.SMEM((NBINS,), jnp.int32),
            pltpu.SMEM((NBINS,), jnp.int32),
        ],
    )
    return fn(query_tokens, rag_scores, attention_mask)
